# Initial kernel scaffold; baseline (speedup 1.0000x reference)
import functools

import jax
import jax.numpy as jnp
from jax import lax
from jax.experimental import pallas as pl
from jax.experimental.pallas import tpu as pltpu
from jax.experimental.pallas import tpu_sc as plsc

N = 10000
E = 160000
D = 256
H = 128
ED = 16
G = 128
T = 12

CH = 128
EP = 163840
NCH = EP // CH
CPT = NCH // 16
ECH_PER_SC = NCH // 2
ECH_PT = ECH_PER_SC // 16
TRASH = N
ACC_ROWS = N + 16
ROWS_PT = N // 16

MB = 1000
NBLK = N // MB

_mesh = plsc.VectorSubcoreMesh(core_axis_name="c", subcore_axis_name="s")
f32 = jnp.float32


@functools.partial(
    pl.kernel,
    out_type=(jax.ShapeDtypeStruct((N, ED), f32),
              jax.ShapeDtypeStruct((N, ED), f32)),
    mesh=_mesh,
    scratch_types=[
        pltpu.VMEM_SHARED((ACC_ROWS, ED), f32),
        pltpu.VMEM((ECH_PT, CH), jnp.int32),
        pltpu.VMEM((CH, ED), f32),
    ],
)
def _esum_kernel(ea_hbm, dst_hbm, z_hbm, e0_hbm, e1_hbm, acc, didx, ebuf):
    c = lax.axis_index("c")
    s = lax.axis_index("s")
    pltpu.sync_copy(z_hbm.at[pl.ds(s * ROWS_PT, ROWS_PT)],
                    acc.at[pl.ds(s * ROWS_PT, ROWS_PT)])
    base = c * ECH_PER_SC + s * ECH_PT
    pltpu.sync_copy(dst_hbm.at[pl.ds(base, ECH_PT)], didx)
    plsc.subcore_barrier()

    def body(j, carry):
        chunk = base + j
        pltpu.sync_copy(ea_hbm.at[pl.ds(chunk * CH, CH)], ebuf)
        pltpu.sync_copy(ebuf, acc.at[didx.at[j]], add=True)
        return carry

    lax.fori_loop(0, ECH_PT, body, 0)
    plsc.subcore_barrier()

    @pl.when(c == 0)
    def _():
        pltpu.sync_copy(acc.at[pl.ds(s * ROWS_PT, ROWS_PT)],
                        e0_hbm.at[pl.ds(s * ROWS_PT, ROWS_PT)])

    @pl.when(c == 1)
    def _():
        pltpu.sync_copy(acc.at[pl.ds(s * ROWS_PT, ROWS_PT)],
                        e1_hbm.at[pl.ds(s * ROWS_PT, ROWS_PT)])


@functools.partial(
    pl.kernel,
    out_type=(jax.ShapeDtypeStruct((N, H), f32),
              jax.ShapeDtypeStruct((N, H), f32)),
    mesh=_mesh,
    scratch_types=[
        pltpu.VMEM_SHARED((ACC_ROWS, H), f32),
        pltpu.VMEM((CPT, CH), jnp.int32),
        pltpu.VMEM((CPT, CH), jnp.int32),
        pltpu.VMEM((CH, H), f32),
        pltpu.SemaphoreType.DMA,
    ],
)
def _spmm_kernel(h0_hbm, h1_hbm, src_hbm, dst_hbm, z_hbm, a0_hbm, a1_hbm,
                 acc, sidx, didx, rows, sem):
    c = lax.axis_index("c")
    s = lax.axis_index("s")
    pltpu.sync_copy(z_hbm.at[pl.ds(s * ROWS_PT, ROWS_PT)],
                    acc.at[pl.ds(s * ROWS_PT, ROWS_PT)])
    pltpu.sync_copy(src_hbm.at[pl.ds(s * CPT, CPT)], sidx)
    pltpu.sync_copy(dst_hbm.at[pl.ds(s * CPT, CPT)], didx)
    plsc.subcore_barrier()

    def do(h_hbm):
        def body(j, carry):
            pltpu.async_copy(h_hbm.at[sidx.at[j]], rows, sem).wait()
            pltpu.sync_copy(rows, acc.at[didx.at[j]], add=True)
            return carry
        lax.fori_loop(0, CPT, body, 0)

    @pl.when(c == 0)
    def _():
        do(h0_hbm)

    @pl.when(c == 1)
    def _():
        do(h1_hbm)

    plsc.subcore_barrier()

    @pl.when(c == 0)
    def _():
        pltpu.sync_copy(acc.at[pl.ds(s * ROWS_PT, ROWS_PT)],
                        a0_hbm.at[pl.ds(s * ROWS_PT, ROWS_PT)])

    @pl.when(c == 1)
    def _():
        pltpu.sync_copy(acc.at[pl.ds(s * ROWS_PT, ROWS_PT)],
                        a1_hbm.at[pl.ds(s * ROWS_PT, ROWS_PT)])


def _update_body(h0, h1, a0, a1, e0, e1, we, w, b, o0, o1):
    es = e0[...] + e1[...]
    e = jnp.dot(es, we[...], preferred_element_type=f32)
    u0 = h0[...] + a0[...] + e[:, :H]
    u1 = h1[...] + a1[...] + e[:, H:]
    u = jnp.concatenate([u0, u1], axis=1)
    hn = jnp.maximum(jnp.dot(u, w[...], preferred_element_type=f32) + b[...], 0.0)
    o0[...] = hn[:, :H]
    o1[...] = hn[:, H:]


_update_call = pl.pallas_call(
    _update_body,
    grid=(NBLK,),
    in_specs=[
        pl.BlockSpec((MB, H), lambda i: (i, 0)),
        pl.BlockSpec((MB, H), lambda i: (i, 0)),
        pl.BlockSpec((MB, H), lambda i: (i, 0)),
        pl.BlockSpec((MB, H), lambda i: (i, 0)),
        pl.BlockSpec((MB, ED), lambda i: (i, 0)),
        pl.BlockSpec((MB, ED), lambda i: (i, 0)),
        pl.BlockSpec((ED, D), lambda i: (0, 0)),
        pl.BlockSpec((D, D), lambda i: (0, 0)),
        pl.BlockSpec((1, D), lambda i: (0, 0)),
    ],
    out_specs=[
        pl.BlockSpec((MB, H), lambda i: (i, 0)),
        pl.BlockSpec((MB, H), lambda i: (i, 0)),
    ],
    out_shape=(jax.ShapeDtypeStruct((N, H), f32),
               jax.ShapeDtypeStruct((N, H), f32)),
)


def _pool_body(h0, h1, bt, wp, bp, out, sums, cnts):
    i = pl.program_id(0)

    @pl.when(i == 0)
    def _():
        sums[...] = jnp.zeros_like(sums)
        cnts[...] = jnp.zeros_like(cnts)

    b = bt[0]
    oh = (lax.broadcasted_iota(jnp.int32, (G, MB), 0) == b).astype(f32)
    h = jnp.concatenate([h0[...], h1[...]], axis=1)
    sums[...] += jnp.dot(oh, h, preferred_element_type=f32)
    cnts[...] += jnp.dot(oh, jnp.ones((MB, G), f32), preferred_element_type=f32)

    @pl.when(i == NBLK - 1)
    def _():
        rec = 1.0 / jnp.maximum(cnts[...], 1.0)
        gr0 = sums[:, :H] * rec
        gr1 = sums[:, H:] * rec
        out[...] = (jnp.dot(gr0, wp[:H, :], preferred_element_type=f32)
                    + jnp.dot(gr1, wp[H:, :], preferred_element_type=f32)
                    + bp[...])


_pool_call = pl.pallas_call(
    _pool_body,
    grid=(NBLK,),
    in_specs=[
        pl.BlockSpec((MB, H), lambda i: (i, 0)),
        pl.BlockSpec((MB, H), lambda i: (i, 0)),
        pl.BlockSpec((1, 1, MB), lambda i: (i, 0, 0)),
        pl.BlockSpec((D, T), lambda i: (0, 0)),
        pl.BlockSpec((1, T), lambda i: (0, 0)),
    ],
    out_specs=pl.BlockSpec((G, T), lambda i: (0, 0)),
    out_shape=jax.ShapeDtypeStruct((G, T), f32),
    scratch_shapes=[pltpu.VMEM((G, D), f32), pltpu.VMEM((G, G), f32)],
)


def kernel(x, edge_index, edge_attr, batch, W_edge, W1, b1, W2, b2, Wp, bp):
    src = edge_index[0].astype(jnp.int32)
    dst = edge_index[1].astype(jnp.int32)
    pad = EP - E
    src2d = jnp.pad(src, (0, pad)).reshape(NCH, CH)
    dst2d = jnp.pad(dst, (0, pad), constant_values=TRASH).reshape(NCH, CH)
    ea_pad = jnp.pad(edge_attr, ((0, pad), (0, 0)))
    zeros_h = jnp.zeros((N, H), f32)
    zeros_e = jnp.zeros((N, ED), f32)
    bt3 = batch.astype(jnp.int32).reshape(NBLK, 1, MB)

    x0 = x[:, :H]
    x1 = x[:, H:]

    e0, e1 = _esum_kernel(ea_pad, dst2d, zeros_e)

    a0, a1 = _spmm_kernel(x0, x1, src2d, dst2d, zeros_h)
    h0, h1 = _update_call(x0, x1, a0, a1, e0, e1, W_edge, W1, b1.reshape(1, D))

    a0, a1 = _spmm_kernel(h0, h1, src2d, dst2d, zeros_h)
    h0, h1 = _update_call(h0, h1, a0, a1, e0, e1, W_edge, W2, b2.reshape(1, D))

    out = _pool_call(h0, h1, bt3, Wp, bp.reshape(1, T))
    node_representation = jnp.concatenate([h0, h1], axis=1)
    return (out, node_representation)


# speedup vs baseline: 2.6756x; 2.6756x over previous
import functools

import jax
import jax.numpy as jnp
from jax import lax
from jax.experimental import pallas as pl
from jax.experimental.pallas import tpu as pltpu
from jax.experimental.pallas import tpu_sc as plsc

N = 10000
E = 160000
D = 256
H = 128
ED = 16
G = 128
T = 12

CH = 128
EP = 163840
NCH = EP // CH
CPT = NCH // 16
ECH_PER_SC = NCH // 2
ECH_PT = ECH_PER_SC // 16
TRASH = N
ACC_ROWS = N + 16
RPT = 624
REM = N - 16 * RPT


def _copy_rows(src, dst, s):
    pltpu.sync_copy(src.at[pl.ds(s * RPT, RPT)], dst.at[pl.ds(s * RPT, RPT)])

    @pl.when(s == 15)
    def _():
        pltpu.sync_copy(src.at[pl.ds(16 * RPT, REM)],
                        dst.at[pl.ds(16 * RPT, REM)])

MB = 1000
NBLK = N // MB

_mesh = plsc.VectorSubcoreMesh(core_axis_name="c", subcore_axis_name="s")
f32 = jnp.float32


@functools.partial(
    pl.kernel,
    out_type=(jax.ShapeDtypeStruct((N, H), f32),
              jax.ShapeDtypeStruct((N, H), f32)),
    mesh=_mesh,
    scratch_types=[
        pltpu.VMEM_SHARED((ACC_ROWS, H), f32),
        pltpu.VMEM((ECH_PT, CH), jnp.int32),
        pltpu.VMEM((CH, H), f32),
    ],
)
def _esum_kernel(ea_hbm, dst_hbm, z_hbm, e0_hbm, e1_hbm, acc, didx, ebuf):
    c = lax.axis_index("c")
    s = lax.axis_index("s")
    _copy_rows(z_hbm, acc, s)
    base = c * ECH_PER_SC + s * ECH_PT
    pltpu.sync_copy(dst_hbm.at[pl.ds(base, ECH_PT)], didx)
    plsc.subcore_barrier()

    def body(j, carry):
        chunk = base + j
        pltpu.sync_copy(ea_hbm.at[pl.ds(chunk * CH, CH)], ebuf)
        pltpu.sync_copy(ebuf, acc.at[didx.at[j]], add=True)
        return carry

    lax.fori_loop(0, ECH_PT, body, 0)
    plsc.subcore_barrier()

    @pl.when(c == 0)
    def _():
        _copy_rows(acc, e0_hbm, s)

    @pl.when(c == 1)
    def _():
        _copy_rows(acc, e1_hbm, s)


@functools.partial(
    pl.kernel,
    out_type=(jax.ShapeDtypeStruct((N, H), f32),
              jax.ShapeDtypeStruct((N, H), f32)),
    mesh=_mesh,
    scratch_types=[
        pltpu.VMEM_SHARED((ACC_ROWS, H), f32),
        pltpu.VMEM((CPT, CH), jnp.int32),
        pltpu.VMEM((CPT, CH), jnp.int32),
        pltpu.VMEM((CH, H), f32),
        pltpu.SemaphoreType.DMA,
    ],
)
def _spmm_kernel(h0_hbm, h1_hbm, src_hbm, dst_hbm, z_hbm, a0_hbm, a1_hbm,
                 acc, sidx, didx, rows, sem):
    c = lax.axis_index("c")
    s = lax.axis_index("s")
    _copy_rows(z_hbm, acc, s)
    pltpu.sync_copy(src_hbm.at[pl.ds(s * CPT, CPT)], sidx)
    pltpu.sync_copy(dst_hbm.at[pl.ds(s * CPT, CPT)], didx)
    plsc.subcore_barrier()

    def do(h_hbm):
        def body(j, carry):
            pltpu.async_copy(h_hbm.at[sidx.at[j]], rows, sem).wait()
            pltpu.sync_copy(rows, acc.at[didx.at[j]], add=True)
            return carry
        lax.fori_loop(0, CPT, body, 0)

    @pl.when(c == 0)
    def _():
        do(h0_hbm)

    @pl.when(c == 1)
    def _():
        do(h1_hbm)

    plsc.subcore_barrier()

    @pl.when(c == 0)
    def _():
        _copy_rows(acc, a0_hbm, s)

    @pl.when(c == 1)
    def _():
        _copy_rows(acc, a1_hbm, s)


def _update_body(h0, h1, a0, a1, e0, e1, we, w, b, o0, o1):
    es = e0[...] + e1[...]
    e = jnp.dot(es, we[...], preferred_element_type=f32)
    u0 = h0[...] + a0[...] + e[:, :H]
    u1 = h1[...] + a1[...] + e[:, H:]
    u = jnp.concatenate([u0, u1], axis=1)
    hn = jnp.maximum(jnp.dot(u, w[...], preferred_element_type=f32) + b[...], 0.0)
    o0[...] = hn[:, :H]
    o1[...] = hn[:, H:]


_update_call = pl.pallas_call(
    _update_body,
    grid=(NBLK,),
    in_specs=[
        pl.BlockSpec((MB, H), lambda i: (i, 0)),
        pl.BlockSpec((MB, H), lambda i: (i, 0)),
        pl.BlockSpec((MB, H), lambda i: (i, 0)),
        pl.BlockSpec((MB, H), lambda i: (i, 0)),
        pl.BlockSpec((MB, H), lambda i: (i, 0)),
        pl.BlockSpec((MB, H), lambda i: (i, 0)),
        pl.BlockSpec((H, D), lambda i: (0, 0)),
        pl.BlockSpec((D, D), lambda i: (0, 0)),
        pl.BlockSpec((1, D), lambda i: (0, 0)),
    ],
    out_specs=[
        pl.BlockSpec((MB, H), lambda i: (i, 0)),
        pl.BlockSpec((MB, H), lambda i: (i, 0)),
    ],
    out_shape=(jax.ShapeDtypeStruct((N, H), f32),
               jax.ShapeDtypeStruct((N, H), f32)),
)


def _pool_body(h0, h1, bt, wp, bp, out, sums, cnts):
    i = pl.program_id(0)

    @pl.when(i == 0)
    def _():
        sums[...] = jnp.zeros_like(sums)
        cnts[...] = jnp.zeros_like(cnts)

    b = bt[0]
    oh = (lax.broadcasted_iota(jnp.int32, (G, MB), 0) == b).astype(f32)
    h = jnp.concatenate([h0[...], h1[...]], axis=1)
    sums[...] += jnp.dot(oh, h, preferred_element_type=f32)
    cnts[...] += jnp.dot(oh, jnp.ones((MB, G), f32), preferred_element_type=f32)

    @pl.when(i == NBLK - 1)
    def _():
        rec = 1.0 / jnp.maximum(cnts[...], 1.0)
        gr0 = sums[:, :H] * rec
        gr1 = sums[:, H:] * rec
        out[...] = (jnp.dot(gr0, wp[:H, :], preferred_element_type=f32)
                    + jnp.dot(gr1, wp[H:, :], preferred_element_type=f32)
                    + bp[...])


_pool_call = pl.pallas_call(
    _pool_body,
    grid=(NBLK,),
    in_specs=[
        pl.BlockSpec((MB, H), lambda i: (i, 0)),
        pl.BlockSpec((MB, H), lambda i: (i, 0)),
        pl.BlockSpec((1, 1, MB), lambda i: (i, 0, 0)),
        pl.BlockSpec((D, T), lambda i: (0, 0)),
        pl.BlockSpec((1, T), lambda i: (0, 0)),
    ],
    out_specs=pl.BlockSpec((G, T), lambda i: (0, 0)),
    out_shape=jax.ShapeDtypeStruct((G, T), f32),
    scratch_shapes=[pltpu.VMEM((G, D), f32), pltpu.VMEM((G, G), f32)],
)


def kernel(x, edge_index, edge_attr, batch, W_edge, W1, b1, W2, b2, Wp, bp):
    src = edge_index[0].astype(jnp.int32)
    dst = edge_index[1].astype(jnp.int32)
    pad = EP - E
    src2d = jnp.pad(src, (0, pad)).reshape(NCH, CH)
    dst2d = jnp.pad(dst, (0, pad), constant_values=TRASH).reshape(NCH, CH)
    ea_pad = jnp.pad(edge_attr, ((0, pad), (0, H - ED)))
    we_pad = jnp.pad(W_edge, ((0, H - ED), (0, 0)))
    zeros_h = jnp.zeros((N, H), f32)
    bt3 = batch.astype(jnp.int32).reshape(NBLK, 1, MB)

    x0 = x[:, :H]
    x1 = x[:, H:]

    e0, e1 = _esum_kernel(ea_pad, dst2d, zeros_h)

    a0, a1 = _spmm_kernel(x0, x1, src2d, dst2d, zeros_h)
    h0, h1 = _update_call(x0, x1, a0, a1, e0, e1, we_pad, W1, b1.reshape(1, D))

    a0, a1 = _spmm_kernel(h0, h1, src2d, dst2d, zeros_h)
    h0, h1 = _update_call(h0, h1, a0, a1, e0, e1, we_pad, W2, b2.reshape(1, D))

    out = _pool_call(h0, h1, bt3, Wp, bp.reshape(1, T))
    node_representation = jnp.concatenate([h0, h1], axis=1)
    return (out, node_representation)


# speedup vs baseline: 2.9621x; 1.1071x over previous
import functools

import jax
import jax.numpy as jnp
from jax import lax
from jax.experimental import pallas as pl
from jax.experimental.pallas import tpu as pltpu
from jax.experimental.pallas import tpu_sc as plsc

N = 10000
E = 160000
D = 256
H = 128
ED = 16
G = 128
T = 12

CH = 128
EP = 163840
NCH = EP // CH
CPT = NCH // 16
IW = 16
ECH_PER_SC = NCH // 2
ECH_PT = ECH_PER_SC // 16
TRASH = N
ACC_ROWS = N + 16
RPT = 624
REM = N - 16 * RPT


def _copy_rows(src, dst, s):
    pltpu.sync_copy(src.at[pl.ds(s * RPT, RPT)], dst.at[pl.ds(s * RPT, RPT)])

    @pl.when(s == 15)
    def _():
        pltpu.sync_copy(src.at[pl.ds(16 * RPT, REM)],
                        dst.at[pl.ds(16 * RPT, REM)])

MB = 1000
NBLK = N // MB

_mesh = plsc.VectorSubcoreMesh(core_axis_name="c", subcore_axis_name="s")
f32 = jnp.float32


@functools.partial(
    pl.kernel,
    out_type=(jax.ShapeDtypeStruct((N, H), f32),
              jax.ShapeDtypeStruct((N, H), f32)),
    mesh=_mesh,
    scratch_types=[
        pltpu.VMEM_SHARED((ACC_ROWS, H), f32),
        pltpu.VMEM((ECH_PT, CH), jnp.int32),
        pltpu.VMEM((2, CH, H), f32),
        pltpu.SemaphoreType.DMA,
    ],
)
def _esum_kernel(ea_hbm, dst_hbm, z_hbm, e0_hbm, e1_hbm, acc, didx, ebuf, sem):
    c = lax.axis_index("c")
    s = lax.axis_index("s")
    _copy_rows(z_hbm, acc, s)
    base = c * ECH_PER_SC + s * ECH_PT
    pltpu.sync_copy(dst_hbm.at[pl.ds(base, ECH_PT)], didx)
    plsc.subcore_barrier()

    pltpu.async_copy(ea_hbm.at[pl.ds(base * CH, CH)], ebuf.at[0], sem)

    def grp(g, carry):
        for b in range(2):
            j = 2 * g + b
            pltpu.make_async_copy(ea_hbm.at[pl.ds((base + j) * CH, CH)],
                                  ebuf.at[b], sem).wait()

            @pl.when(j + 1 < ECH_PT)
            def _():
                pltpu.async_copy(ea_hbm.at[pl.ds((base + j + 1) * CH, CH)],
                                 ebuf.at[1 - b], sem)

            pltpu.sync_copy(ebuf.at[b], acc.at[didx.at[j]], add=True)
        return carry

    lax.fori_loop(0, ECH_PT // 2, grp, 0)
    plsc.subcore_barrier()

    @pl.when(c == 0)
    def _():
        _copy_rows(acc, e0_hbm, s)

    @pl.when(c == 1)
    def _():
        _copy_rows(acc, e1_hbm, s)


@functools.partial(
    pl.kernel,
    out_type=(jax.ShapeDtypeStruct((N, H), f32),
              jax.ShapeDtypeStruct((N, H), f32)),
    mesh=_mesh,
    scratch_types=[
        pltpu.VMEM_SHARED((ACC_ROWS, H), f32),
        pltpu.VMEM((IW, CH), jnp.int32),
        pltpu.VMEM((IW, CH), jnp.int32),
        pltpu.VMEM((2, CH, H), f32),
        pltpu.SemaphoreType.DMA,
    ],
)
def _spmm_kernel(h0_hbm, h1_hbm, src_hbm, dst_hbm, z_hbm, a0_hbm, a1_hbm,
                 acc, sidx, didx, rows, sem):
    c = lax.axis_index("c")
    s = lax.axis_index("s")
    _copy_rows(z_hbm, acc, s)
    plsc.subcore_barrier()

    def do(h_hbm):
        def win(w, carry):
            base = s * CPT + w * IW
            pltpu.sync_copy(src_hbm.at[pl.ds(base, IW)], sidx)
            pltpu.sync_copy(dst_hbm.at[pl.ds(base, IW)], didx)
            pltpu.async_copy(h_hbm.at[sidx.at[0]], rows.at[0], sem)

            def grp(g, carry2):
                for b in range(2):
                    j = 2 * g + b
                    pltpu.make_async_copy(h_hbm.at[sidx.at[j]],
                                          rows.at[b], sem).wait()

                    @pl.when(j + 1 < IW)
                    def _():
                        pltpu.async_copy(h_hbm.at[sidx.at[j + 1]],
                                         rows.at[1 - b], sem)

                    pltpu.sync_copy(rows.at[b], acc.at[didx.at[j]], add=True)
                return carry2

            lax.fori_loop(0, IW // 2, grp, 0)
            return carry

        lax.fori_loop(0, CPT // IW, win, 0)

    @pl.when(c == 0)
    def _():
        do(h0_hbm)

    @pl.when(c == 1)
    def _():
        do(h1_hbm)

    plsc.subcore_barrier()

    @pl.when(c == 0)
    def _():
        _copy_rows(acc, a0_hbm, s)

    @pl.when(c == 1)
    def _():
        _copy_rows(acc, a1_hbm, s)


def _update_body(h0, h1, a0, a1, e0, e1, we, w, b, o0, o1):
    es = e0[...] + e1[...]
    e = jnp.dot(es, we[...], preferred_element_type=f32)
    u0 = h0[...] + a0[...] + e[:, :H]
    u1 = h1[...] + a1[...] + e[:, H:]
    u = jnp.concatenate([u0, u1], axis=1)
    hn = jnp.maximum(jnp.dot(u, w[...], preferred_element_type=f32) + b[...], 0.0)
    o0[...] = hn[:, :H]
    o1[...] = hn[:, H:]


_update_call = pl.pallas_call(
    _update_body,
    grid=(NBLK,),
    in_specs=[
        pl.BlockSpec((MB, H), lambda i: (i, 0)),
        pl.BlockSpec((MB, H), lambda i: (i, 0)),
        pl.BlockSpec((MB, H), lambda i: (i, 0)),
        pl.BlockSpec((MB, H), lambda i: (i, 0)),
        pl.BlockSpec((MB, H), lambda i: (i, 0)),
        pl.BlockSpec((MB, H), lambda i: (i, 0)),
        pl.BlockSpec((H, D), lambda i: (0, 0)),
        pl.BlockSpec((D, D), lambda i: (0, 0)),
        pl.BlockSpec((1, D), lambda i: (0, 0)),
    ],
    out_specs=[
        pl.BlockSpec((MB, H), lambda i: (i, 0)),
        pl.BlockSpec((MB, H), lambda i: (i, 0)),
    ],
    out_shape=(jax.ShapeDtypeStruct((N, H), f32),
               jax.ShapeDtypeStruct((N, H), f32)),
)


def _pool_body(h0, h1, bt, wp, bp, out, sums, cnts):
    i = pl.program_id(0)

    @pl.when(i == 0)
    def _():
        sums[...] = jnp.zeros_like(sums)
        cnts[...] = jnp.zeros_like(cnts)

    b = bt[0]
    oh = (lax.broadcasted_iota(jnp.int32, (G, MB), 0) == b).astype(f32)
    h = jnp.concatenate([h0[...], h1[...]], axis=1)
    sums[...] += jnp.dot(oh, h, preferred_element_type=f32)
    cnts[...] += jnp.dot(oh, jnp.ones((MB, G), f32), preferred_element_type=f32)

    @pl.when(i == NBLK - 1)
    def _():
        rec = 1.0 / jnp.maximum(cnts[...], 1.0)
        gr0 = sums[:, :H] * rec
        gr1 = sums[:, H:] * rec
        out[...] = (jnp.dot(gr0, wp[:H, :], preferred_element_type=f32)
                    + jnp.dot(gr1, wp[H:, :], preferred_element_type=f32)
                    + bp[...])


_pool_call = pl.pallas_call(
    _pool_body,
    grid=(NBLK,),
    in_specs=[
        pl.BlockSpec((MB, H), lambda i: (i, 0)),
        pl.BlockSpec((MB, H), lambda i: (i, 0)),
        pl.BlockSpec((1, 1, MB), lambda i: (i, 0, 0)),
        pl.BlockSpec((D, T), lambda i: (0, 0)),
        pl.BlockSpec((1, T), lambda i: (0, 0)),
    ],
    out_specs=pl.BlockSpec((G, T), lambda i: (0, 0)),
    out_shape=jax.ShapeDtypeStruct((G, T), f32),
    scratch_shapes=[pltpu.VMEM((G, D), f32), pltpu.VMEM((G, G), f32)],
)


def kernel(x, edge_index, edge_attr, batch, W_edge, W1, b1, W2, b2, Wp, bp):
    src = edge_index[0].astype(jnp.int32)
    dst = edge_index[1].astype(jnp.int32)
    pad = EP - E
    src2d = jnp.pad(src, (0, pad)).reshape(NCH, CH)
    dst2d = jnp.pad(dst, (0, pad), constant_values=TRASH).reshape(NCH, CH)
    ea_pad = jnp.pad(edge_attr, ((0, pad), (0, H - ED)))
    we_pad = jnp.pad(W_edge, ((0, H - ED), (0, 0)))
    zeros_h = jnp.zeros((N, H), f32)
    bt3 = batch.astype(jnp.int32).reshape(NBLK, 1, MB)

    x0 = x[:, :H]
    x1 = x[:, H:]

    e0, e1 = _esum_kernel(ea_pad, dst2d, zeros_h)

    a0, a1 = _spmm_kernel(x0, x1, src2d, dst2d, zeros_h)
    h0, h1 = _update_call(x0, x1, a0, a1, e0, e1, we_pad, W1, b1.reshape(1, D))

    a0, a1 = _spmm_kernel(h0, h1, src2d, dst2d, zeros_h)
    h0, h1 = _update_call(h0, h1, a0, a1, e0, e1, we_pad, W2, b2.reshape(1, D))

    out = _pool_call(h0, h1, bt3, Wp, bp.reshape(1, T))
    node_representation = jnp.concatenate([h0, h1], axis=1)
    return (out, node_representation)
